# baseline (speedup 1.0000x reference)
import functools

import jax
import jax.numpy as jnp
from jax import lax
from jax.experimental import pallas as pl
from jax.experimental.pallas import tpu as pltpu
from jax.experimental.pallas import tpu_sc as plsc

N = 10000
E = 160000
H = 256
EO = 128
NO = 128
GO = 128

BN = 1000
BE = 2000

_lrelu = lambda t: jnp.where(t >= 0, t, 0.2 * t)
_relu = lambda t: jnp.maximum(t, 0.0)


def _mm(a, b):
    return jax.lax.dot_general(a, b, (((1,), (0,)), ((), ())),
                               preferred_element_type=jnp.float32)


def _tc_a_body(x_ref, wn0_ref, bn0_ref, wn1_ref, bn1_ref, ws_ref, wt_ref,
               wc_ref, bg0_ref, wg1_ref, bg1_ref, wu_ref, be1_ref, un_ref,
               bnn_ref, x1lo_ref, x1hi_ref, xs_ref, xt_ref, xclo_ref,
               xchi_ref, u1_ref, cu0_ref, ub_ref):
    x = x_ref[...]
    h = _lrelu(_mm(x, wn0_ref[...]) + bn0_ref[...])
    x1 = _lrelu(_mm(h, wn1_ref[...]) + bn1_ref[...])
    x1lo_ref[...] = x1[:, :128]
    x1hi_ref[...] = x1[:, 128:]
    xs_ref[...] = _mm(x1, ws_ref[...])
    xt_ref[...] = _mm(x1, wt_ref[...])
    xc = _mm(x1, wc_ref[...])
    xclo_ref[...] = xc[:, :128]
    xchi_ref[...] = xc[:, 128:]

    @pl.when(pl.program_id(0) == 0)
    def _():
        g1 = _lrelu(jnp.broadcast_to(bg0_ref[...], (8, 256)))
        u1 = _lrelu(_mm(g1, wg1_ref[...]) + bg1_ref[...])
        u1_ref[...] = u1
        cu0_ref[...] = _mm(u1, wu_ref[...]) + be1_ref[...]
        ub_ref[...] = _mm(u1, un_ref[...]) + bnn_ref[...]


def _tc_a(x, wn0, bn0, wn1, bn1, ws, wt, wc, bg0, wg1, bg1, wu, be1, un, bnn):
    nb = N // BN
    full = lambda s: pl.BlockSpec(s, lambda i: (0, 0))
    return pl.pallas_call(
        _tc_a_body,
        grid=(nb,),
        in_specs=[
            pl.BlockSpec((BN, 128), lambda i: (i, 0)),
            full((128, 256)), full((1, 256)), full((256, 256)), full((1, 256)),
            full((256, 256)), full((256, 256)), full((256, 256)),
            full((1, 256)), full((256, 256)), full((1, 256)),
            full((256, 256)), full((1, 256)), full((256, 256)), full((1, 256)),
        ],
        out_specs=[
            pl.BlockSpec((BN, 128), lambda i: (i, 0)),
            pl.BlockSpec((BN, 128), lambda i: (i, 0)),
            pl.BlockSpec((BN, 256), lambda i: (i, 0)),
            pl.BlockSpec((BN, 256), lambda i: (i, 0)),
            pl.BlockSpec((BN, 128), lambda i: (i, 0)),
            pl.BlockSpec((BN, 128), lambda i: (i, 0)),
            pl.BlockSpec((8, 256), lambda i: (0, 0)),
            pl.BlockSpec((8, 256), lambda i: (0, 0)),
            pl.BlockSpec((8, 256), lambda i: (0, 0)),
        ],
        out_shape=[
            jax.ShapeDtypeStruct((N, 128), jnp.float32),
            jax.ShapeDtypeStruct((N, 128), jnp.float32),
            jax.ShapeDtypeStruct((N, 256), jnp.float32),
            jax.ShapeDtypeStruct((N, 256), jnp.float32),
            jax.ShapeDtypeStruct((N, 128), jnp.float32),
            jax.ShapeDtypeStruct((N, 128), jnp.float32),
            jax.ShapeDtypeStruct((8, 256), jnp.float32),
            jax.ShapeDtypeStruct((8, 256), jnp.float32),
            jax.ShapeDtypeStruct((8, 256), jnp.float32),
        ],
    )(x, wn0, bn0, wn1, bn1, ws, wt, wc, bg0, wg1, bg1, wu, be1, un, bnn)


def _tc_b_body(e_ref, gsum_ref, we0_ref, be0_ref, we1_ref, beb_ref,
               we_ref, cu0_ref, w2_ref, b2_ref, w3_ref, b3_ref, wa_ref,
               e2_ref, zplo_ref, zphi_ref, sume2_ref):
    h = _lrelu(_mm(e_ref[...], we0_ref[...]) + be0_ref[...])
    e1 = _lrelu(_mm(h, we1_ref[...]) + beb_ref[...])
    h1 = _lrelu(_mm(e1, we_ref[...]) + gsum_ref[...] + cu0_ref[0:1, :])
    h2 = _lrelu(_mm(h1, w2_ref[...]) + b2_ref[...])
    e2 = _relu(_mm(h2, w3_ref[...]) + b3_ref[...])
    e2_ref[...] = e2
    zp = _mm(e2, wa_ref[...])
    zplo_ref[...] = zp[:, :128]
    zphi_ref[...] = zp[:, 128:]
    s = jnp.sum(e2, axis=0, keepdims=True)

    @pl.when(pl.program_id(0) == 0)
    def _():
        sume2_ref[...] = jnp.zeros_like(sume2_ref)

    sume2_ref[...] += jnp.broadcast_to(s, (8, 128))


def _tc_b(e, gsum, we0, be0, we1, beb, we, cu0, w2, b2, w3, b3, wa):
    nb = E // BE
    full = lambda s: pl.BlockSpec(s, lambda i: (0, 0))
    return pl.pallas_call(
        _tc_b_body,
        grid=(nb,),
        in_specs=[
            pl.BlockSpec((BE, 16), lambda i: (i, 0)),
            pl.BlockSpec((BE, 256), lambda i: (i, 0)),
            full((16, 256)), full((1, 256)), full((256, 256)), full((1, 256)),
            full((256, 256)), full((8, 256)),
            full((256, 256)), full((1, 256)), full((256, 128)), full((1, 128)),
            full((128, 256)),
        ],
        out_specs=[
            pl.BlockSpec((BE, 128), lambda i: (i, 0)),
            pl.BlockSpec((BE, 128), lambda i: (i, 0)),
            pl.BlockSpec((BE, 128), lambda i: (i, 0)),
            pl.BlockSpec((8, 128), lambda i: (0, 0)),
        ],
        out_shape=[
            jax.ShapeDtypeStruct((E, 128), jnp.float32),
            jax.ShapeDtypeStruct((E, 128), jnp.float32),
            jax.ShapeDtypeStruct((E, 128), jnp.float32),
            jax.ShapeDtypeStruct((8, 128), jnp.float32),
        ],
    )(e, gsum, we0, be0, we1, beb, we, cu0, w2, b2, w3, b3, wa)


def _tc_d_body(segzlo_ref, segzhi_ref, cnt_ref,
               x1lo_ref, x1hi_ref, ub_ref, sume2_ref, u1_ref,
               ba_ref, bb_ref,
               w2_ref, b2_ref, w3_ref, b3_ref,
               wg0a_ref, wg0b_ref, wg0c_ref, bg0_ref, wg1_ref, bg1_ref,
               wg2_ref, bg2_ref, qu_ref, b1e_ref, qs_ref, qt_ref,
               xs2_ref, xt2_ref, cu1_ref, sumx2_ref):
    cnt = jnp.sum(cnt_ref[...], axis=1)[:, None]
    inv = 1.0 / jnp.maximum(cnt, 1.0)
    pos = (cnt > 0).astype(jnp.float32)
    t = jnp.concatenate([segzlo_ref[...], segzhi_ref[...]], axis=1)
    t = t * inv
    t = t + (_mm(x1lo_ref[...], ba_ref[...])
             + _mm(x1hi_ref[...], bb_ref[...])) * pos
    n1 = _lrelu(t + ub_ref[0:1, :])
    n2 = _lrelu(_mm(n1, w2_ref[...]) + b2_ref[...])
    x2 = _relu(_mm(n2, w3_ref[...]) + b3_ref[...])
    xs2_ref[...] = _mm(x2, qs_ref[...])
    xt2_ref[...] = _mm(x2, qt_ref[...])
    s = jnp.sum(x2, axis=0, keepdims=True)

    @pl.when(pl.program_id(0) == 0)
    def _():
        sumx2_ref[...] = jnp.zeros_like(sumx2_ref)

    sumx2_ref[...] += jnp.broadcast_to(s, (8, 128))

    @pl.when(pl.program_id(0) == pl.num_programs(0) - 1)
    def _():
        g1 = _lrelu(_mm(sumx2_ref[...], wg0a_ref[...])
                    + _mm(sume2_ref[...], wg0b_ref[...])
                    + _mm(u1_ref[...], wg0c_ref[...]) + bg0_ref[...])
        g2 = _lrelu(_mm(g1, wg1_ref[...]) + bg1_ref[...])
        u2 = _relu(_mm(g2, wg2_ref[...]) + bg2_ref[...])
        cu1_ref[...] = _mm(u2, qu_ref[...]) + b1e_ref[...]


def _tc_d(segzlo, segzhi, cnt_all, x1lo, x1hi, ub, sume2, u1,
          ba, bb, w2, b2, w3, b3,
          wg0a, wg0b, wg0c, bg0, wg1, bg1, wg2, bg2, qu, b1e, qs, qt):
    nb = N // BN
    full = lambda s: pl.BlockSpec(s, lambda i: (0, 0))
    return pl.pallas_call(
        _tc_d_body,
        grid=(nb,),
        in_specs=[
            pl.BlockSpec((BN, 128), lambda i: (i, 0)),
            pl.BlockSpec((BN, 128), lambda i: (i, 0)),
            pl.BlockSpec((BN, 32), lambda i: (i, 0)),
            pl.BlockSpec((BN, 128), lambda i: (i, 0)),
            pl.BlockSpec((BN, 128), lambda i: (i, 0)),
            full((8, 256)), full((8, 128)), full((8, 256)),
            full((128, 256)), full((128, 256)),
            full((256, 256)), full((1, 256)), full((256, 128)), full((1, 128)),
            full((128, 256)), full((128, 256)), full((256, 256)), full((1, 256)),
            full((256, 256)), full((1, 256)), full((256, 128)), full((1, 128)),
            full((128, 128)), full((1, 128)), full((128, 128)), full((128, 128)),
        ],
        out_specs=[
            pl.BlockSpec((BN, 128), lambda i: (i, 0)),
            pl.BlockSpec((BN, 128), lambda i: (i, 0)),
            pl.BlockSpec((8, 128), lambda i: (0, 0)),
            pl.BlockSpec((8, 128), lambda i: (0, 0)),
        ],
        out_shape=[
            jax.ShapeDtypeStruct((N, 128), jnp.float32),
            jax.ShapeDtypeStruct((N, 128), jnp.float32),
            jax.ShapeDtypeStruct((8, 128), jnp.float32),
            jax.ShapeDtypeStruct((8, 128), jnp.float32),
        ],
    )(segzlo, segzhi, cnt_all, x1lo, x1hi, ub, sume2, u1,
      ba, bb, w2, b2, w3, b3,
      wg0a, wg0b, wg0c, bg0, wg1, bg1, wg2, bg2, qu, b1e, qs, qt)


def _tc_f_body(e2_ref, gsum2_ref, qe_ref, cu1_ref, out_ref):
    e3 = _relu(_mm(e2_ref[...], qe_ref[...])
               + gsum2_ref[...] + cu1_ref[0:1, :])
    m = jnp.max(e3, axis=-1, keepdims=True)
    p = jnp.exp(e3 - m)
    out_ref[...] = p / jnp.sum(p, axis=-1, keepdims=True)


def _tc_f(e2, gsum2, qe, cu1):
    nb = E // BE
    full = lambda s: pl.BlockSpec(s, lambda i: (0, 0))
    return pl.pallas_call(
        _tc_f_body,
        grid=(nb,),
        in_specs=[
            pl.BlockSpec((BE, 128), lambda i: (i, 0)),
            pl.BlockSpec((BE, 128), lambda i: (i, 0)),
            full((128, 128)), full((8, 128)),
        ],
        out_specs=[pl.BlockSpec((BE, 128), lambda i: (i, 0))],
        out_shape=[jax.ShapeDtypeStruct((E, 128), jnp.float32)],
    )(e2, gsum2, qe, cu1)[0]


_CE = 40


def _sc_gather_pair(table_a, table_b, src2d, tgt2d, width):
    info = plsc.get_sparse_core_info()
    nw = info.num_cores * info.num_subcores
    ew = E // nw
    ce = _CE
    nch = ew // ce
    nvec = width // 16
    mesh = plsc.VectorSubcoreMesh(core_axis_name="c", subcore_axis_name="s")

    @functools.partial(
        pl.kernel,
        out_type=jax.ShapeDtypeStruct((E, width), jnp.float32),
        mesh=mesh,
        compiler_params=pltpu.CompilerParams(needs_layout_passes=False),
        scratch_types=[
            pltpu.VMEM((128, ce), jnp.int32),
            pltpu.VMEM((128, ce), jnp.int32),
            pltpu.VMEM((2, ce, width), jnp.float32),
            pltpu.VMEM((2, ce, width), jnp.float32),
            pltpu.SemaphoreType.DMA,
        ],
    )
    def k(ta_hbm, tb_hbm, src_hbm, tgt_hbm, out_hbm, sidx, tidx, bufa, bufb,
          sem):
        wid = lax.axis_index("s") * info.num_cores + lax.axis_index("c")
        base = wid * ew
        pltpu.sync_copy(src_hbm.at[wid], sidx)
        pltpu.sync_copy(tgt_hbm.at[wid], tidx)
        pltpu.async_copy(ta_hbm.at[sidx.at[0]], bufa.at[0], sem)
        pltpu.async_copy(tb_hbm.at[tidx.at[0]], bufb.at[0], sem)

        def chunk(k_, _):
            p = lax.rem(k_, 2)
            pltpu.make_async_copy(ta_hbm.at[pl.ds(0, ce)], bufa.at[p], sem).wait()
            pltpu.make_async_copy(tb_hbm.at[pl.ds(0, ce)], bufb.at[p], sem).wait()

            @pl.when(k_ + 1 < nch)
            def _():
                q = lax.rem(k_ + 1, 2)
                pltpu.async_copy(ta_hbm.at[sidx.at[k_ + 1]], bufa.at[q], sem)
                pltpu.async_copy(tb_hbm.at[tidx.at[k_ + 1]], bufb.at[q], sem)

            def row(r_, _):
                for j in range(nvec):
                    bufa[p, r_, pl.ds(j * 16, 16)] = (
                        bufa[p, r_, pl.ds(j * 16, 16)]
                        + bufb[p, r_, pl.ds(j * 16, 16)])
                return ()

            lax.fori_loop(0, ce, row, (), unroll=False)
            pltpu.sync_copy(bufa.at[p], out_hbm.at[pl.ds(base + k_ * ce, ce)])
            return ()

        lax.fori_loop(0, nch, chunk, (), unroll=False)

    return k(table_a, table_b, src2d, tgt2d)


NC = 10016


def _sc_gather_pair_cnt(table_a, table_b, src2d, tgt2d, width, srcc):
    info = plsc.get_sparse_core_info()
    nw = info.num_cores * info.num_subcores
    ew = E // nw
    ce = _CE
    nch = ew // ce
    nvec = width // 16
    mesh = plsc.VectorSubcoreMesh(core_axis_name="c", subcore_axis_name="s")

    @functools.partial(
        pl.kernel,
        out_type=[
            jax.ShapeDtypeStruct((E, width), jnp.float32),
            jax.ShapeDtypeStruct((nw, NC), jnp.float32),
        ],
        mesh=mesh,
        compiler_params=pltpu.CompilerParams(needs_layout_passes=False),
        scratch_types=[
            pltpu.VMEM((128, ce), jnp.int32),
            pltpu.VMEM((128, ce), jnp.int32),
            pltpu.VMEM((40, 128), jnp.int32),
            pltpu.VMEM((NC,), jnp.float32),
            pltpu.VMEM((2, ce, width), jnp.float32),
            pltpu.VMEM((2, ce, width), jnp.float32),
            pltpu.SemaphoreType.DMA,
        ],
    )
    def k(ta_hbm, tb_hbm, src_hbm, tgt_hbm, srcc_hbm, out_hbm, cnt_hbm,
          sidx, tidx, cidx, cntv, bufa, bufb, sem):
        wid = lax.axis_index("s") * info.num_cores + lax.axis_index("c")
        base = wid * ew
        pltpu.sync_copy(src_hbm.at[wid], sidx)
        pltpu.sync_copy(tgt_hbm.at[wid], tidx)
        pltpu.sync_copy(srcc_hbm.at[wid], cidx)
        pltpu.async_copy(ta_hbm.at[sidx.at[0]], bufa.at[0], sem)
        pltpu.async_copy(tb_hbm.at[tidx.at[0]], bufb.at[0], sem)

        zeros16 = jnp.zeros((16,), jnp.float32)

        def zz(i, _):
            cntv[pl.ds(i * 16, 16)] = zeros16
            return ()

        lax.fori_loop(0, NC // 16, zz, (), unroll=False)
        ones16 = jnp.ones((16,), jnp.float32)

        def crow(r_, _):
            for j in range(8):
                iv = cidx[r_, pl.ds(j * 16, 16)]
                plsc.addupdate_scatter(cntv, [iv], ones16)
            return ()

        lax.fori_loop(0, 40, crow, (), unroll=False)
        pltpu.sync_copy(cntv, cnt_hbm.at[wid])

        def chunk(k_, _):
            p = lax.rem(k_, 2)
            pltpu.make_async_copy(ta_hbm.at[pl.ds(0, ce)], bufa.at[p], sem).wait()
            pltpu.make_async_copy(tb_hbm.at[pl.ds(0, ce)], bufb.at[p], sem).wait()

            @pl.when(k_ + 1 < nch)
            def _():
                q = lax.rem(k_ + 1, 2)
                pltpu.async_copy(ta_hbm.at[sidx.at[k_ + 1]], bufa.at[q], sem)
                pltpu.async_copy(tb_hbm.at[tidx.at[k_ + 1]], bufb.at[q], sem)

            def row(r_, _):
                for j in range(nvec):
                    bufa[p, r_, pl.ds(j * 16, 16)] = (
                        bufa[p, r_, pl.ds(j * 16, 16)]
                        + bufb[p, r_, pl.ds(j * 16, 16)])
                return ()

            lax.fori_loop(0, ce, row, (), unroll=False)
            pltpu.sync_copy(bufa.at[p], out_hbm.at[pl.ds(base + k_ * ce, ce)])
            return ()

        lax.fori_loop(0, nch, chunk, (), unroll=False)

    return k(table_a, table_b, src2d, tgt2d, srcc)


NP = 10112


def _sc_segment_z(zplo, zphi, xclo, xchi, srcst, tgtst, zx):
    info = plsc.get_sparse_core_info()
    ns = info.num_subcores
    ew = E // ns
    ce = 80
    stripe = NP // ns
    mesh = plsc.VectorSubcoreMesh(core_axis_name="c", subcore_axis_name="s")

    nch = ew // ce

    @functools.partial(
        pl.kernel,
        out_type=[
            jax.ShapeDtypeStruct((NP, 128), jnp.float32),
            jax.ShapeDtypeStruct((NP, 128), jnp.float32),
        ],
        mesh=mesh,
        compiler_params=pltpu.CompilerParams(needs_layout_passes=False),
        scratch_types=[
            pltpu.VMEM_SHARED((NP, 128), jnp.float32),
            pltpu.VMEM((2, ce), jnp.int32),
            pltpu.VMEM((2, ce), jnp.int32),
            pltpu.VMEM((2, ce, 128), jnp.float32),
            pltpu.VMEM((2, ce, 128), jnp.float32),
            pltpu.SemaphoreType.DMA,
            pltpu.SemaphoreType.DMA,
            pltpu.SemaphoreType.DMA,
        ],
    )
    def k(zplo_hbm, zphi_hbm, xclo_hbm, xchi_hbm, srcst_hbm, tgtst_hbm,
          zx_hbm, seglo_hbm, seghi_hbm, accz, sidxb, tidxb, bufg, bufz,
          isem, gsem, zsem):
        cid = lax.axis_index("c")
        sid = lax.axis_index("s")
        row0 = sid * stripe
        pltpu.sync_copy(zx_hbm.at[pl.ds(row0, stripe)],
                        accz.at[pl.ds(row0, stripe)])
        plsc.subcore_barrier()
        base = sid * ew

        def body(zp_hbm, xc_hbm):
            pltpu.sync_copy(srcst_hbm.at[sid, 0], sidxb.at[0])
            pltpu.sync_copy(tgtst_hbm.at[sid, 0], tidxb.at[0])
            pltpu.async_copy(xc_hbm.at[tidxb.at[0]], bufg.at[0], gsem)
            pltpu.async_copy(zp_hbm.at[pl.ds(base, ce)], bufz.at[0], zsem)
            pltpu.async_copy(srcst_hbm.at[sid, 1], sidxb.at[1], isem)
            pltpu.async_copy(tgtst_hbm.at[sid, 1], tidxb.at[1], isem)

            def chunk(k_, _):
                p = lax.rem(k_, 2)
                q = lax.rem(k_ + 1, 2)

                @pl.when(k_ + 1 < nch)
                def _():
                    pltpu.make_async_copy(srcst_hbm.at[sid, 0], sidxb.at[q],
                                          isem).wait()
                    pltpu.make_async_copy(tgtst_hbm.at[sid, 0], tidxb.at[q],
                                          isem).wait()
                    pltpu.async_copy(xc_hbm.at[tidxb.at[q]], bufg.at[q], gsem)
                    pltpu.async_copy(zp_hbm.at[pl.ds(base + (k_ + 1) * ce, ce)],
                                     bufz.at[q], zsem)

                pltpu.make_async_copy(xc_hbm.at[pl.ds(0, ce)], bufg.at[p],
                                      gsem).wait()
                pltpu.make_async_copy(zp_hbm.at[pl.ds(0, ce)], bufz.at[p],
                                      zsem).wait()

                def row(r_, _):
                    for j in range(8):
                        bufg[p, r_, pl.ds(j * 16, 16)] = (
                            bufg[p, r_, pl.ds(j * 16, 16)]
                            + bufz[p, r_, pl.ds(j * 16, 16)])
                    return ()

                lax.fori_loop(0, ce, row, (), unroll=False)
                pltpu.sync_copy(bufg.at[p], accz.at[sidxb.at[p]], add=True)

                @pl.when(k_ + 2 < nch)
                def _():
                    pltpu.async_copy(srcst_hbm.at[sid, k_ + 2], sidxb.at[p],
                                     isem)
                    pltpu.async_copy(tgtst_hbm.at[sid, k_ + 2], tidxb.at[p],
                                     isem)

                return ()

            lax.fori_loop(0, nch, chunk, (), unroll=False)

        @pl.when(cid == 0)
        def _():
            body(zplo_hbm, xclo_hbm)

        @pl.when(cid == 1)
        def _():
            body(zphi_hbm, xchi_hbm)

        plsc.subcore_barrier()

        @pl.when(cid == 0)
        def _():
            pltpu.sync_copy(accz.at[pl.ds(row0, stripe)],
                            seglo_hbm.at[pl.ds(row0, stripe)])

        @pl.when(cid == 1)
        def _():
            pltpu.sync_copy(accz.at[pl.ds(row0, stripe)],
                            seghi_hbm.at[pl.ds(row0, stripe)])

    return k(zplo, zphi, xclo, xchi, srcst, tgtst, zx)


def kernel(x, edge_index, e, params):
    f32 = jnp.float32
    src = edge_index[:, 0]
    tgt = edge_index[:, 1]
    r = lambda b: b.reshape(1, -1)

    wn, bn = params["emb_node"]
    we_, be_ = params["emb_edge"]
    wg, bg = params["emb_glob"]
    w0e, b0e = params["gnn0_edge"]
    w0n, b0n = params["gnn0_node"]
    w0g, b0g = params["gnn0_glob"]
    w1e, b1e = params["gnn1_edge"]

    W1 = w0e[0]
    We, Ws, Wt, Wu = W1[0:256], W1[256:512], W1[512:768], W1[768:1024]
    Wn1 = w0n[0]
    A, B, C, U = Wn1[0:128], Wn1[128:384], Wn1[384:640], Wn1[640:896]
    Q = w1e[0]
    Qe, Qs, Qt, Qu = Q[0:128], Q[128:256], Q[256:384], Q[384:512]
    Wg0 = w0g[0]

    def idx3(a, nw, ce, rows):
        a3 = a.reshape(nw, -1, ce)
        return jnp.pad(a3, ((0, 0), (0, rows - a3.shape[1]), (0, 0)))

    src40 = idx3(src, 32, _CE, 128)
    tgt40 = idx3(tgt, 32, _CE, 128)
    srcst = src.reshape(16, -1, 80)
    tgtst = tgt.reshape(16, -1, 80)
    srcp = jnp.pad(src.reshape(32, 5000), ((0, 0), (0, 120)),
                   constant_values=NC - 8).reshape(32, 40, 128)

    x1lo, x1hi, xs, xt, xclo, xchi, u1, cu0, ub = _tc_a(
        x, wn[0], r(bn[0]), wn[1], r(bn[1]), Ws, Wt, C,
        r(bg[0]), wg[1], r(bg[1]), Wu, r(b0e[0]), U, r(b0n[0]))

    gsum, cnt_pad = _sc_gather_pair_cnt(xs, xt, src40, tgt40, 256, srcp)
    cnt_all = cnt_pad[:, :N]

    e2, zplo, zphi, sume2 = _tc_b(
        e, gsum, we_[0], r(be_[0]), we_[1], r(be_[1]), We, cu0,
        w0e[1], r(b0e[1]), w0e[2], r(b0e[2]), A)

    zx = jnp.zeros((NP, 128), f32)
    segzlo, segzhi = _sc_segment_z(zplo, zphi, xclo, xchi, srcst, tgtst, zx)

    xs2, xt2, cu1, _sumx2 = _tc_d(
        segzlo[:N], segzhi[:N], cnt_all.T, x1lo, x1hi, ub, sume2, u1,
        B[:128], B[128:],
        w0n[1], r(b0n[1]), w0n[2], r(b0n[2]),
        Wg0[0:128], Wg0[128:256], Wg0[256:512], r(b0g[0]),
        w0g[1], r(b0g[1]), w0g[2], r(b0g[2]), Qu, r(b1e[0]), Qs, Qt)

    gsum2 = _sc_gather_pair(xs2, xt2, src40, tgt40, 128)

    return _tc_f(e2, gsum2, Qe, cu1)


# speedup vs baseline: 1.4743x; 1.4743x over previous
import functools

import jax
import jax.numpy as jnp
from jax import lax
from jax.experimental import pallas as pl
from jax.experimental.pallas import tpu as pltpu
from jax.experimental.pallas import tpu_sc as plsc

N = 10000
E = 160000
H = 256
EO = 128
NO = 128
GO = 128

BN = 1000
BE = 2000

_lrelu = lambda t: jnp.where(t >= 0, t, 0.2 * t)
_relu = lambda t: jnp.maximum(t, 0.0)


def _mm(a, b):
    return jax.lax.dot_general(a, b, (((1,), (0,)), ((), ())),
                               preferred_element_type=jnp.float32)


def _tc_a_body(x_ref, wn0_ref, bn0_ref, wn1_ref, bn1_ref, ws_ref, wt_ref,
               wc_ref, bg0_ref, wg1_ref, bg1_ref, wu_ref, be1_ref, un_ref,
               bnn_ref, x1lo_ref, x1hi_ref, xs_ref, xt_ref, xclo_ref,
               xchi_ref, u1_ref, cu0_ref, ub_ref):
    x = x_ref[...]
    h = _lrelu(_mm(x, wn0_ref[...]) + bn0_ref[...])
    x1 = _lrelu(_mm(h, wn1_ref[...]) + bn1_ref[...])
    x1lo_ref[...] = x1[:, :128]
    x1hi_ref[...] = x1[:, 128:]
    xs_ref[...] = _mm(x1, ws_ref[...])
    xt_ref[...] = _mm(x1, wt_ref[...])
    xc = _mm(x1, wc_ref[...])
    xclo_ref[...] = xc[:, :128]
    xchi_ref[...] = xc[:, 128:]

    @pl.when(pl.program_id(0) == 0)
    def _():
        g1 = _lrelu(jnp.broadcast_to(bg0_ref[...], (8, 256)))
        u1 = _lrelu(_mm(g1, wg1_ref[...]) + bg1_ref[...])
        u1_ref[...] = u1
        cu0_ref[...] = _mm(u1, wu_ref[...]) + be1_ref[...]
        ub_ref[...] = _mm(u1, un_ref[...]) + bnn_ref[...]


def _tc_a(x, wn0, bn0, wn1, bn1, ws, wt, wc, bg0, wg1, bg1, wu, be1, un, bnn):
    nb = N // BN
    full = lambda s: pl.BlockSpec(s, lambda i: (0, 0))
    return pl.pallas_call(
        _tc_a_body,
        grid=(nb,),
        in_specs=[
            pl.BlockSpec((BN, 128), lambda i: (i, 0)),
            full((128, 256)), full((1, 256)), full((256, 256)), full((1, 256)),
            full((256, 256)), full((256, 256)), full((256, 256)),
            full((1, 256)), full((256, 256)), full((1, 256)),
            full((256, 256)), full((1, 256)), full((256, 256)), full((1, 256)),
        ],
        out_specs=[
            pl.BlockSpec((BN, 128), lambda i: (i, 0)),
            pl.BlockSpec((BN, 128), lambda i: (i, 0)),
            pl.BlockSpec((BN, 256), lambda i: (i, 0)),
            pl.BlockSpec((BN, 256), lambda i: (i, 0)),
            pl.BlockSpec((BN, 128), lambda i: (i, 0)),
            pl.BlockSpec((BN, 128), lambda i: (i, 0)),
            pl.BlockSpec((8, 256), lambda i: (0, 0)),
            pl.BlockSpec((8, 256), lambda i: (0, 0)),
            pl.BlockSpec((8, 256), lambda i: (0, 0)),
        ],
        out_shape=[
            jax.ShapeDtypeStruct((N, 128), jnp.float32),
            jax.ShapeDtypeStruct((N, 128), jnp.float32),
            jax.ShapeDtypeStruct((N, 256), jnp.float32),
            jax.ShapeDtypeStruct((N, 256), jnp.float32),
            jax.ShapeDtypeStruct((N, 128), jnp.float32),
            jax.ShapeDtypeStruct((N, 128), jnp.float32),
            jax.ShapeDtypeStruct((8, 256), jnp.float32),
            jax.ShapeDtypeStruct((8, 256), jnp.float32),
            jax.ShapeDtypeStruct((8, 256), jnp.float32),
        ],
    )(x, wn0, bn0, wn1, bn1, ws, wt, wc, bg0, wg1, bg1, wu, be1, un, bnn)


def _tc_b_body(e_ref, gsum_ref, we0_ref, be0_ref, we1_ref, beb_ref,
               we_ref, cu0_ref, w2_ref, b2_ref, w3_ref, b3_ref,
               e2_ref, sume2_ref):
    h = _lrelu(_mm(e_ref[...], we0_ref[...]) + be0_ref[...])
    e1 = _lrelu(_mm(h, we1_ref[...]) + beb_ref[...])
    h1 = _lrelu(_mm(e1, we_ref[...]) + gsum_ref[...] + cu0_ref[0:1, :])
    h2 = _lrelu(_mm(h1, w2_ref[...]) + b2_ref[...])
    e2 = _relu(_mm(h2, w3_ref[...]) + b3_ref[...])
    e2_ref[...] = e2
    s = jnp.sum(e2, axis=0, keepdims=True)

    @pl.when(pl.program_id(0) == 0)
    def _():
        sume2_ref[...] = jnp.zeros_like(sume2_ref)

    sume2_ref[...] += jnp.broadcast_to(s, (8, 128))


def _tc_b(e, gsum, we0, be0, we1, beb, we, cu0, w2, b2, w3, b3):
    nb = E // BE
    full = lambda s: pl.BlockSpec(s, lambda i: (0, 0))
    return pl.pallas_call(
        _tc_b_body,
        grid=(nb,),
        in_specs=[
            pl.BlockSpec((BE, 16), lambda i: (i, 0)),
            pl.BlockSpec((BE, 256), lambda i: (i, 0)),
            full((16, 256)), full((1, 256)), full((256, 256)), full((1, 256)),
            full((256, 256)), full((8, 256)),
            full((256, 256)), full((1, 256)), full((256, 128)), full((1, 128)),
        ],
        out_specs=[
            pl.BlockSpec((BE, 128), lambda i: (i, 0)),
            pl.BlockSpec((8, 128), lambda i: (0, 0)),
        ],
        out_shape=[
            jax.ShapeDtypeStruct((E, 128), jnp.float32),
            jax.ShapeDtypeStruct((8, 128), jnp.float32),
        ],
    )(e, gsum, we0, be0, we1, beb, we, cu0, w2, b2, w3, b3)


def _tc_d_body(sege0_ref, sege1_ref, segzlo_ref, segzhi_ref, cnt_ref,
               x1lo_ref, x1hi_ref, ub_ref, sume2_ref, u1_ref,
               a_ref, ba_ref, bb_ref,
               w2_ref, b2_ref, w3_ref, b3_ref,
               wg0a_ref, wg0b_ref, wg0c_ref, bg0_ref, wg1_ref, bg1_ref,
               wg2_ref, bg2_ref, qu_ref, b1e_ref, qs_ref, qt_ref,
               xs2_ref, xt2_ref, cu1_ref, sumx2_ref):
    cnt = jnp.sum(cnt_ref[...], axis=1)[:, None]
    inv = 1.0 / jnp.maximum(cnt, 1.0)
    pos = (cnt > 0).astype(jnp.float32)
    t = (_mm(sege0_ref[...] + sege1_ref[...], a_ref[...])
         + jnp.concatenate([segzlo_ref[...], segzhi_ref[...]], axis=1))
    t = t * inv
    t = t + (_mm(x1lo_ref[...], ba_ref[...])
             + _mm(x1hi_ref[...], bb_ref[...])) * pos
    n1 = _lrelu(t + ub_ref[0:1, :])
    n2 = _lrelu(_mm(n1, w2_ref[...]) + b2_ref[...])
    x2 = _relu(_mm(n2, w3_ref[...]) + b3_ref[...])
    xs2_ref[...] = _mm(x2, qs_ref[...])
    xt2_ref[...] = _mm(x2, qt_ref[...])
    s = jnp.sum(x2, axis=0, keepdims=True)

    @pl.when(pl.program_id(0) == 0)
    def _():
        sumx2_ref[...] = jnp.zeros_like(sumx2_ref)

    sumx2_ref[...] += jnp.broadcast_to(s, (8, 128))

    @pl.when(pl.program_id(0) == pl.num_programs(0) - 1)
    def _():
        g1 = _lrelu(_mm(sumx2_ref[...], wg0a_ref[...])
                    + _mm(sume2_ref[...], wg0b_ref[...])
                    + _mm(u1_ref[...], wg0c_ref[...]) + bg0_ref[...])
        g2 = _lrelu(_mm(g1, wg1_ref[...]) + bg1_ref[...])
        u2 = _relu(_mm(g2, wg2_ref[...]) + bg2_ref[...])
        cu1_ref[...] = _mm(u2, qu_ref[...]) + b1e_ref[...]


def _tc_d(sege0, sege1, segzlo, segzhi, cnt_all, x1lo, x1hi, ub, sume2, u1,
          a, ba, bb, w2, b2, w3, b3,
          wg0a, wg0b, wg0c, bg0, wg1, bg1, wg2, bg2, qu, b1e, qs, qt):
    nb = N // BN
    full = lambda s: pl.BlockSpec(s, lambda i: (0, 0))
    return pl.pallas_call(
        _tc_d_body,
        grid=(nb,),
        in_specs=[
            pl.BlockSpec((BN, 128), lambda i: (i, 0)),
            pl.BlockSpec((BN, 128), lambda i: (i, 0)),
            pl.BlockSpec((BN, 128), lambda i: (i, 0)),
            pl.BlockSpec((BN, 128), lambda i: (i, 0)),
            pl.BlockSpec((BN, 32), lambda i: (i, 0)),
            pl.BlockSpec((BN, 128), lambda i: (i, 0)),
            pl.BlockSpec((BN, 128), lambda i: (i, 0)),
            full((8, 256)), full((8, 128)), full((8, 256)),
            full((128, 256)), full((128, 256)), full((128, 256)),
            full((256, 256)), full((1, 256)), full((256, 128)), full((1, 128)),
            full((128, 256)), full((128, 256)), full((256, 256)), full((1, 256)),
            full((256, 256)), full((1, 256)), full((256, 128)), full((1, 128)),
            full((128, 128)), full((1, 128)), full((128, 128)), full((128, 128)),
        ],
        out_specs=[
            pl.BlockSpec((BN, 128), lambda i: (i, 0)),
            pl.BlockSpec((BN, 128), lambda i: (i, 0)),
            pl.BlockSpec((8, 128), lambda i: (0, 0)),
            pl.BlockSpec((8, 128), lambda i: (0, 0)),
        ],
        out_shape=[
            jax.ShapeDtypeStruct((N, 128), jnp.float32),
            jax.ShapeDtypeStruct((N, 128), jnp.float32),
            jax.ShapeDtypeStruct((8, 128), jnp.float32),
            jax.ShapeDtypeStruct((8, 128), jnp.float32),
        ],
    )(sege0, sege1, segzlo, segzhi, cnt_all, x1lo, x1hi, ub, sume2, u1,
      a, ba, bb, w2, b2, w3, b3,
      wg0a, wg0b, wg0c, bg0, wg1, bg1, wg2, bg2, qu, b1e, qs, qt)


def _tc_f_body(e2_ref, gsum2_ref, qe_ref, cu1_ref, out_ref):
    e3 = _relu(_mm(e2_ref[...], qe_ref[...])
               + gsum2_ref[...] + cu1_ref[0:1, :])
    m = jnp.max(e3, axis=-1, keepdims=True)
    p = jnp.exp(e3 - m)
    out_ref[...] = p / jnp.sum(p, axis=-1, keepdims=True)


def _tc_f(e2, gsum2, qe, cu1):
    nb = E // BE
    full = lambda s: pl.BlockSpec(s, lambda i: (0, 0))
    return pl.pallas_call(
        _tc_f_body,
        grid=(nb,),
        in_specs=[
            pl.BlockSpec((BE, 128), lambda i: (i, 0)),
            pl.BlockSpec((BE, 128), lambda i: (i, 0)),
            full((128, 128)), full((8, 128)),
        ],
        out_specs=[pl.BlockSpec((BE, 128), lambda i: (i, 0))],
        out_shape=[jax.ShapeDtypeStruct((E, 128), jnp.float32)],
    )(e2, gsum2, qe, cu1)[0]


_CE = 40


def _sc_gather_pair(table_a, table_b, src2d, tgt2d, width):
    info = plsc.get_sparse_core_info()
    nw = info.num_cores * info.num_subcores
    ew = E // nw
    ce = _CE
    nch = ew // ce
    nvec = width // 16
    mesh = plsc.VectorSubcoreMesh(core_axis_name="c", subcore_axis_name="s")

    @functools.partial(
        pl.kernel,
        out_type=jax.ShapeDtypeStruct((E, width), jnp.float32),
        mesh=mesh,
        compiler_params=pltpu.CompilerParams(needs_layout_passes=False),
        scratch_types=[
            pltpu.VMEM((128, ce), jnp.int32),
            pltpu.VMEM((128, ce), jnp.int32),
            pltpu.VMEM((2, ce, width), jnp.float32),
            pltpu.VMEM((2, ce, width), jnp.float32),
            pltpu.SemaphoreType.DMA,
        ],
    )
    def k(ta_hbm, tb_hbm, src_hbm, tgt_hbm, out_hbm, sidx, tidx, bufa, bufb,
          sem):
        wid = lax.axis_index("s") * info.num_cores + lax.axis_index("c")
        base = wid * ew
        pltpu.sync_copy(src_hbm.at[wid], sidx)
        pltpu.sync_copy(tgt_hbm.at[wid], tidx)
        pltpu.async_copy(ta_hbm.at[sidx.at[0]], bufa.at[0], sem)
        pltpu.async_copy(tb_hbm.at[tidx.at[0]], bufb.at[0], sem)

        def chunk(k_, _):
            p = lax.rem(k_, 2)
            pltpu.make_async_copy(ta_hbm.at[pl.ds(0, ce)], bufa.at[p], sem).wait()
            pltpu.make_async_copy(tb_hbm.at[pl.ds(0, ce)], bufb.at[p], sem).wait()

            @pl.when(k_ + 1 < nch)
            def _():
                q = lax.rem(k_ + 1, 2)
                pltpu.async_copy(ta_hbm.at[sidx.at[k_ + 1]], bufa.at[q], sem)
                pltpu.async_copy(tb_hbm.at[tidx.at[k_ + 1]], bufb.at[q], sem)

            def row(r_, _):
                for j in range(nvec):
                    bufa[p, r_, pl.ds(j * 16, 16)] = (
                        bufa[p, r_, pl.ds(j * 16, 16)]
                        + bufb[p, r_, pl.ds(j * 16, 16)])
                return ()

            lax.fori_loop(0, ce, row, (), unroll=False)
            pltpu.sync_copy(bufa.at[p], out_hbm.at[pl.ds(base + k_ * ce, ce)])
            return ()

        lax.fori_loop(0, nch, chunk, (), unroll=False)

    return k(table_a, table_b, src2d, tgt2d)


NC = 10016


def _sc_gather_pair_cnt(table_a, table_b, src2d, tgt2d, width, srcc):
    info = plsc.get_sparse_core_info()
    nw = info.num_cores * info.num_subcores
    ew = E // nw
    ce = _CE
    nch = ew // ce
    nvec = width // 16
    mesh = plsc.VectorSubcoreMesh(core_axis_name="c", subcore_axis_name="s")

    @functools.partial(
        pl.kernel,
        out_type=[
            jax.ShapeDtypeStruct((E, width), jnp.float32),
            jax.ShapeDtypeStruct((nw, NC), jnp.float32),
        ],
        mesh=mesh,
        compiler_params=pltpu.CompilerParams(needs_layout_passes=False),
        scratch_types=[
            pltpu.VMEM((128, ce), jnp.int32),
            pltpu.VMEM((128, ce), jnp.int32),
            pltpu.VMEM((40, 128), jnp.int32),
            pltpu.VMEM((NC,), jnp.float32),
            pltpu.VMEM((2, ce, width), jnp.float32),
            pltpu.VMEM((2, ce, width), jnp.float32),
            pltpu.SemaphoreType.DMA,
        ],
    )
    def k(ta_hbm, tb_hbm, src_hbm, tgt_hbm, srcc_hbm, out_hbm, cnt_hbm,
          sidx, tidx, cidx, cntv, bufa, bufb, sem):
        wid = lax.axis_index("s") * info.num_cores + lax.axis_index("c")
        base = wid * ew
        pltpu.sync_copy(src_hbm.at[wid], sidx)
        pltpu.sync_copy(tgt_hbm.at[wid], tidx)
        pltpu.sync_copy(srcc_hbm.at[wid], cidx)
        pltpu.async_copy(ta_hbm.at[sidx.at[0]], bufa.at[0], sem)
        pltpu.async_copy(tb_hbm.at[tidx.at[0]], bufb.at[0], sem)

        zeros16 = jnp.zeros((16,), jnp.float32)

        def zz(i, _):
            cntv[pl.ds(i * 16, 16)] = zeros16
            return ()

        lax.fori_loop(0, NC // 16, zz, (), unroll=False)
        ones16 = jnp.ones((16,), jnp.float32)

        def crow(r_, _):
            for j in range(8):
                iv = cidx[r_, pl.ds(j * 16, 16)]
                plsc.addupdate_scatter(cntv, [iv], ones16)
            return ()

        lax.fori_loop(0, 40, crow, (), unroll=False)
        pltpu.sync_copy(cntv, cnt_hbm.at[wid])

        def chunk(k_, _):
            p = lax.rem(k_, 2)
            pltpu.make_async_copy(ta_hbm.at[pl.ds(0, ce)], bufa.at[p], sem).wait()
            pltpu.make_async_copy(tb_hbm.at[pl.ds(0, ce)], bufb.at[p], sem).wait()

            @pl.when(k_ + 1 < nch)
            def _():
                q = lax.rem(k_ + 1, 2)
                pltpu.async_copy(ta_hbm.at[sidx.at[k_ + 1]], bufa.at[q], sem)
                pltpu.async_copy(tb_hbm.at[tidx.at[k_ + 1]], bufb.at[q], sem)

            def row(r_, _):
                for j in range(nvec):
                    bufa[p, r_, pl.ds(j * 16, 16)] = (
                        bufa[p, r_, pl.ds(j * 16, 16)]
                        + bufb[p, r_, pl.ds(j * 16, 16)])
                return ()

            lax.fori_loop(0, ce, row, (), unroll=False)
            pltpu.sync_copy(bufa.at[p], out_hbm.at[pl.ds(base + k_ * ce, ce)])
            return ()

        lax.fori_loop(0, nch, chunk, (), unroll=False)

    return k(table_a, table_b, src2d, tgt2d, srcc)


NP = 10112


def _sc_segment_xc(xclo, xchi, srcst, tgtst, zx):
    info = plsc.get_sparse_core_info()
    ns = info.num_subcores
    ew = E // ns
    ce = 80
    stripe = NP // ns
    mesh = plsc.VectorSubcoreMesh(core_axis_name="c", subcore_axis_name="s")

    nch = ew // ce

    @functools.partial(
        pl.kernel,
        out_type=[
            jax.ShapeDtypeStruct((NP, 128), jnp.float32),
            jax.ShapeDtypeStruct((NP, 128), jnp.float32),
        ],
        mesh=mesh,
        compiler_params=pltpu.CompilerParams(needs_layout_passes=False),
        scratch_types=[
            pltpu.VMEM_SHARED((NP, 128), jnp.float32),
            pltpu.VMEM((2, ce), jnp.int32),
            pltpu.VMEM((2, ce), jnp.int32),
            pltpu.VMEM((2, ce, 128), jnp.float32),
            pltpu.SemaphoreType.DMA,
            pltpu.SemaphoreType.DMA,
        ],
    )
    def k(xclo_hbm, xchi_hbm, srcst_hbm, tgtst_hbm, zx_hbm,
          seglo_hbm, seghi_hbm, accz, sidxb, tidxb, bufg, isem, gsem):
        cid = lax.axis_index("c")
        sid = lax.axis_index("s")
        row0 = sid * stripe
        pltpu.sync_copy(zx_hbm.at[pl.ds(row0, stripe)],
                        accz.at[pl.ds(row0, stripe)])
        plsc.subcore_barrier()

        def body(xc_hbm):
            pltpu.sync_copy(srcst_hbm.at[sid, 0], sidxb.at[0])
            pltpu.sync_copy(tgtst_hbm.at[sid, 0], tidxb.at[0])
            pltpu.async_copy(xc_hbm.at[tidxb.at[0]], bufg.at[0], gsem)
            pltpu.async_copy(srcst_hbm.at[sid, 1], sidxb.at[1], isem)
            pltpu.async_copy(tgtst_hbm.at[sid, 1], tidxb.at[1], isem)

            def chunk(k_, _):
                p = lax.rem(k_, 2)
                q = lax.rem(k_ + 1, 2)

                @pl.when(k_ + 1 < nch)
                def _():
                    pltpu.make_async_copy(srcst_hbm.at[sid, 0], sidxb.at[q],
                                          isem).wait()
                    pltpu.make_async_copy(tgtst_hbm.at[sid, 0], tidxb.at[q],
                                          isem).wait()
                    pltpu.async_copy(xc_hbm.at[tidxb.at[q]], bufg.at[q], gsem)

                pltpu.make_async_copy(xc_hbm.at[pl.ds(0, ce)], bufg.at[p],
                                      gsem).wait()
                pltpu.sync_copy(bufg.at[p], accz.at[sidxb.at[p]], add=True)

                @pl.when(k_ + 2 < nch)
                def _():
                    pltpu.async_copy(srcst_hbm.at[sid, k_ + 2], sidxb.at[p],
                                     isem)
                    pltpu.async_copy(tgtst_hbm.at[sid, k_ + 2], tidxb.at[p],
                                     isem)

                return ()

            lax.fori_loop(0, nch, chunk, (), unroll=False)

        @pl.when(cid == 0)
        def _():
            body(xclo_hbm)

        @pl.when(cid == 1)
        def _():
            body(xchi_hbm)

        plsc.subcore_barrier()

        @pl.when(cid == 0)
        def _():
            pltpu.sync_copy(accz.at[pl.ds(row0, stripe)],
                            seglo_hbm.at[pl.ds(row0, stripe)])

        @pl.when(cid == 1)
        def _():
            pltpu.sync_copy(accz.at[pl.ds(row0, stripe)],
                            seghi_hbm.at[pl.ds(row0, stripe)])

    return k(xclo, xchi, srcst, tgtst, zx)


def _sc_segment_e(e2, src, zx):
    info = plsc.get_sparse_core_info()
    nc, ns = info.num_cores, info.num_subcores
    ew = E // (nc * ns)
    ce = _CE
    stripe = NP // ns
    mesh = plsc.VectorSubcoreMesh(core_axis_name="c", subcore_axis_name="s")

    nch = ew // ce

    @functools.partial(
        pl.kernel,
        out_type=[
            jax.ShapeDtypeStruct((NP, 128), jnp.float32),
            jax.ShapeDtypeStruct((NP, 128), jnp.float32),
        ],
        mesh=mesh,
        compiler_params=pltpu.CompilerParams(needs_layout_passes=False),
        scratch_types=[
            pltpu.VMEM_SHARED((NP, 128), jnp.float32),
            pltpu.VMEM((128, ce), jnp.int32),
            pltpu.VMEM((2, ce, 128), jnp.float32),
            pltpu.SemaphoreType.DMA,
        ],
    )
    def k(e2_hbm, src_hbm, zx_hbm, sege0_hbm, sege1_hbm,
          acce, sidx, bufe, sem):
        cid = lax.axis_index("c")
        sid = lax.axis_index("s")
        row0 = sid * stripe
        pltpu.sync_copy(zx_hbm.at[pl.ds(row0, stripe)],
                        acce.at[pl.ds(row0, stripe)])
        wid = cid * ns + sid
        pltpu.sync_copy(src_hbm.at[wid], sidx)
        plsc.subcore_barrier()

        base = wid * ew
        pltpu.async_copy(e2_hbm.at[pl.ds(base, ce)], bufe.at[0], sem)

        def chunk(k_, _):
            p = lax.rem(k_, 2)
            pltpu.make_async_copy(e2_hbm.at[pl.ds(0, ce)], bufe.at[p],
                                  sem).wait()

            @pl.when(k_ + 1 < nch)
            def _():
                q = lax.rem(k_ + 1, 2)
                pltpu.async_copy(e2_hbm.at[pl.ds(base + (k_ + 1) * ce, ce)],
                                 bufe.at[q], sem)

            pltpu.sync_copy(bufe.at[p], acce.at[sidx.at[k_]], add=True)
            return ()

        lax.fori_loop(0, nch, chunk, (), unroll=False)
        plsc.subcore_barrier()

        @pl.when(cid == 0)
        def _():
            pltpu.sync_copy(acce.at[pl.ds(row0, stripe)],
                            sege0_hbm.at[pl.ds(row0, stripe)])

        @pl.when(cid == 1)
        def _():
            pltpu.sync_copy(acce.at[pl.ds(row0, stripe)],
                            sege1_hbm.at[pl.ds(row0, stripe)])

    return k(e2, src, zx)


def kernel(x, edge_index, e, params):
    f32 = jnp.float32
    src = edge_index[:, 0]
    tgt = edge_index[:, 1]
    r = lambda b: b.reshape(1, -1)

    wn, bn = params["emb_node"]
    we_, be_ = params["emb_edge"]
    wg, bg = params["emb_glob"]
    w0e, b0e = params["gnn0_edge"]
    w0n, b0n = params["gnn0_node"]
    w0g, b0g = params["gnn0_glob"]
    w1e, b1e = params["gnn1_edge"]

    W1 = w0e[0]
    We, Ws, Wt, Wu = W1[0:256], W1[256:512], W1[512:768], W1[768:1024]
    Wn1 = w0n[0]
    A, B, C, U = Wn1[0:128], Wn1[128:384], Wn1[384:640], Wn1[640:896]
    Q = w1e[0]
    Qe, Qs, Qt, Qu = Q[0:128], Q[128:256], Q[256:384], Q[384:512]
    Wg0 = w0g[0]

    def idx3(a, nw, ce, rows):
        a3 = a.reshape(nw, -1, ce)
        return jnp.pad(a3, ((0, 0), (0, rows - a3.shape[1]), (0, 0)))

    src40 = idx3(src, 32, _CE, 128)
    tgt40 = idx3(tgt, 32, _CE, 128)
    srcst = src.reshape(16, -1, 80)
    tgtst = tgt.reshape(16, -1, 80)
    srcp = jnp.pad(src.reshape(32, 5000), ((0, 0), (0, 120)),
                   constant_values=NC - 8).reshape(32, 40, 128)

    x1lo, x1hi, xs, xt, xclo, xchi, u1, cu0, ub = _tc_a(
        x, wn[0], r(bn[0]), wn[1], r(bn[1]), Ws, Wt, C,
        r(bg[0]), wg[1], r(bg[1]), Wu, r(b0e[0]), U, r(b0n[0]))

    gsum, cnt_pad = _sc_gather_pair_cnt(xs, xt, src40, tgt40, 256, srcp)
    cnt_all = cnt_pad[:, :N]

    zx = jnp.zeros((NP, 128), f32)
    segzlo, segzhi = _sc_segment_xc(xclo, xchi, srcst, tgtst, zx)

    e2, sume2 = _tc_b(
        e, gsum, we_[0], r(be_[0]), we_[1], r(be_[1]), We, cu0,
        w0e[1], r(b0e[1]), w0e[2], r(b0e[2]))

    sege0, sege1 = _sc_segment_e(e2, src40, zx)

    xs2, xt2, cu1, _sumx2 = _tc_d(
        sege0[:N], sege1[:N], segzlo[:N], segzhi[:N], cnt_all.T,
        x1lo, x1hi, ub, sume2, u1,
        A, B[:128], B[128:],
        w0n[1], r(b0n[1]), w0n[2], r(b0n[2]),
        Wg0[0:128], Wg0[128:256], Wg0[256:512], r(b0g[0]),
        w0g[1], r(b0g[1]), w0g[2], r(b0g[2]), Qu, r(b1e[0]), Qs, Qt)

    gsum2 = _sc_gather_pair(xs2, xt2, src40, tgt40, 128)

    return _tc_f(e2, gsum2, Qe, cu1)


# speedup vs baseline: 1.5080x; 1.0228x over previous
import functools

import jax
import jax.numpy as jnp
from jax import lax
from jax.experimental import pallas as pl
from jax.experimental.pallas import tpu as pltpu
from jax.experimental.pallas import tpu_sc as plsc

N = 10000
E = 160000
H = 256
EO = 128
NO = 128
GO = 128

BN = 1000
BE = 2000

_lrelu = lambda t: jnp.where(t >= 0, t, 0.2 * t)
_relu = lambda t: jnp.maximum(t, 0.0)


def _mm(a, b):
    return jax.lax.dot_general(a, b, (((1,), (0,)), ((), ())),
                               preferred_element_type=jnp.float32)


def _tc_a_body(x_ref, wn0_ref, bn0_ref, wn1_ref, bn1_ref, ws_ref, wt_ref,
               wc_ref, bg0_ref, wg1_ref, bg1_ref, wu_ref, be1_ref, un_ref,
               bnn_ref, x1lo_ref, x1hi_ref, xs_ref, xt_ref, xclo_ref,
               xchi_ref, u1_ref, cu0_ref, ub_ref):
    x = x_ref[...]
    h = _lrelu(_mm(x, wn0_ref[...]) + bn0_ref[...])
    x1 = _lrelu(_mm(h, wn1_ref[...]) + bn1_ref[...])
    x1lo_ref[...] = x1[:, :128]
    x1hi_ref[...] = x1[:, 128:]
    xs_ref[...] = _mm(x1, ws_ref[...])
    xt_ref[...] = _mm(x1, wt_ref[...])
    xc = _mm(x1, wc_ref[...])
    xclo_ref[...] = xc[:, :128]
    xchi_ref[...] = xc[:, 128:]

    @pl.when(pl.program_id(0) == 0)
    def _():
        g1 = _lrelu(jnp.broadcast_to(bg0_ref[...], (8, 256)))
        u1 = _lrelu(_mm(g1, wg1_ref[...]) + bg1_ref[...])
        u1_ref[...] = u1
        cu0_ref[...] = _mm(u1, wu_ref[...]) + be1_ref[...]
        ub_ref[...] = _mm(u1, un_ref[...]) + bnn_ref[...]


def _tc_a(x, wn0, bn0, wn1, bn1, ws, wt, wc, bg0, wg1, bg1, wu, be1, un, bnn):
    nb = N // BN
    full = lambda s: pl.BlockSpec(s, lambda i: (0, 0))
    return pl.pallas_call(
        _tc_a_body,
        grid=(nb,),
        in_specs=[
            pl.BlockSpec((BN, 128), lambda i: (i, 0)),
            full((128, 256)), full((1, 256)), full((256, 256)), full((1, 256)),
            full((256, 256)), full((256, 256)), full((256, 256)),
            full((1, 256)), full((256, 256)), full((1, 256)),
            full((256, 256)), full((1, 256)), full((256, 256)), full((1, 256)),
        ],
        out_specs=[
            pl.BlockSpec((BN, 128), lambda i: (i, 0)),
            pl.BlockSpec((BN, 128), lambda i: (i, 0)),
            pl.BlockSpec((BN, 256), lambda i: (i, 0)),
            pl.BlockSpec((BN, 256), lambda i: (i, 0)),
            pl.BlockSpec((BN, 128), lambda i: (i, 0)),
            pl.BlockSpec((BN, 128), lambda i: (i, 0)),
            pl.BlockSpec((8, 256), lambda i: (0, 0)),
            pl.BlockSpec((8, 256), lambda i: (0, 0)),
            pl.BlockSpec((8, 256), lambda i: (0, 0)),
        ],
        out_shape=[
            jax.ShapeDtypeStruct((N, 128), jnp.float32),
            jax.ShapeDtypeStruct((N, 128), jnp.float32),
            jax.ShapeDtypeStruct((N, 256), jnp.float32),
            jax.ShapeDtypeStruct((N, 256), jnp.float32),
            jax.ShapeDtypeStruct((N, 128), jnp.float32),
            jax.ShapeDtypeStruct((N, 128), jnp.float32),
            jax.ShapeDtypeStruct((8, 256), jnp.float32),
            jax.ShapeDtypeStruct((8, 256), jnp.float32),
            jax.ShapeDtypeStruct((8, 256), jnp.float32),
        ],
    )(x, wn0, bn0, wn1, bn1, ws, wt, wc, bg0, wg1, bg1, wu, be1, un, bnn)


def _tc_b_body(e_ref, gsum_ref, we0_ref, be0_ref, we1_ref, beb_ref,
               we_ref, cu0_ref, w2_ref, b2_ref, w3_ref, b3_ref,
               e2_ref, sume2_ref):
    h = _lrelu(_mm(e_ref[...], we0_ref[...]) + be0_ref[...])
    e1 = _lrelu(_mm(h, we1_ref[...]) + beb_ref[...])
    h1 = _lrelu(_mm(e1, we_ref[...]) + gsum_ref[...] + cu0_ref[0:1, :])
    h2 = _lrelu(_mm(h1, w2_ref[...]) + b2_ref[...])
    e2 = _relu(_mm(h2, w3_ref[...]) + b3_ref[...])
    e2_ref[...] = e2
    s = jnp.sum(e2, axis=0, keepdims=True)

    @pl.when(pl.program_id(0) == 0)
    def _():
        sume2_ref[...] = jnp.zeros_like(sume2_ref)

    sume2_ref[...] += jnp.broadcast_to(s, (8, 128))


def _tc_b(e, gsum, we0, be0, we1, beb, we, cu0, w2, b2, w3, b3):
    nb = E // BE
    full = lambda s: pl.BlockSpec(s, lambda i: (0, 0))
    return pl.pallas_call(
        _tc_b_body,
        grid=(nb,),
        in_specs=[
            pl.BlockSpec((BE, 16), lambda i: (i, 0)),
            pl.BlockSpec((BE, 256), lambda i: (i, 0)),
            full((16, 256)), full((1, 256)), full((256, 256)), full((1, 256)),
            full((256, 256)), full((8, 256)),
            full((256, 256)), full((1, 256)), full((256, 128)), full((1, 128)),
        ],
        out_specs=[
            pl.BlockSpec((BE, 128), lambda i: (i, 0)),
            pl.BlockSpec((8, 128), lambda i: (0, 0)),
        ],
        out_shape=[
            jax.ShapeDtypeStruct((E, 128), jnp.float32),
            jax.ShapeDtypeStruct((8, 128), jnp.float32),
        ],
    )(e, gsum, we0, be0, we1, beb, we, cu0, w2, b2, w3, b3)


def _tc_d_body(sege0_ref, sege1_ref, segzlo_ref, segzhi_ref, cnt_ref,
               x1lo_ref, x1hi_ref, ub_ref, sume2_ref, u1_ref,
               a_ref, ba_ref, bb_ref,
               w2_ref, b2_ref, w3_ref, b3_ref,
               wg0a_ref, wg0b_ref, wg0c_ref, bg0_ref, wg1_ref, bg1_ref,
               wg2_ref, bg2_ref, qu_ref, b1e_ref, qs_ref, qt_ref,
               xs2_ref, xt2_ref, cu1_ref, sumx2_ref):
    cnt = jnp.sum(cnt_ref[...], axis=1)[:, None]
    inv = 1.0 / jnp.maximum(cnt, 1.0)
    pos = (cnt > 0).astype(jnp.float32)
    t = (_mm(sege0_ref[...] + sege1_ref[...], a_ref[...])
         + jnp.concatenate([segzlo_ref[...], segzhi_ref[...]], axis=1))
    t = t * inv
    t = t + (_mm(x1lo_ref[...], ba_ref[...])
             + _mm(x1hi_ref[...], bb_ref[...])) * pos
    n1 = _lrelu(t + ub_ref[0:1, :])
    n2 = _lrelu(_mm(n1, w2_ref[...]) + b2_ref[...])
    x2 = _relu(_mm(n2, w3_ref[...]) + b3_ref[...])
    xs2_ref[...] = _mm(x2, qs_ref[...])
    xt2_ref[...] = _mm(x2, qt_ref[...])
    s = jnp.sum(x2, axis=0, keepdims=True)

    @pl.when(pl.program_id(0) == 0)
    def _():
        sumx2_ref[...] = jnp.zeros_like(sumx2_ref)

    sumx2_ref[...] += jnp.broadcast_to(s, (8, 128))

    @pl.when(pl.program_id(0) == pl.num_programs(0) - 1)
    def _():
        g1 = _lrelu(_mm(sumx2_ref[...], wg0a_ref[...])
                    + _mm(sume2_ref[...], wg0b_ref[...])
                    + _mm(u1_ref[...], wg0c_ref[...]) + bg0_ref[...])
        g2 = _lrelu(_mm(g1, wg1_ref[...]) + bg1_ref[...])
        u2 = _relu(_mm(g2, wg2_ref[...]) + bg2_ref[...])
        cu1_ref[...] = _mm(u2, qu_ref[...]) + b1e_ref[...]


def _tc_d(sege0, sege1, segzlo, segzhi, cnt_all, x1lo, x1hi, ub, sume2, u1,
          a, ba, bb, w2, b2, w3, b3,
          wg0a, wg0b, wg0c, bg0, wg1, bg1, wg2, bg2, qu, b1e, qs, qt):
    nb = N // BN
    full = lambda s: pl.BlockSpec(s, lambda i: (0, 0))
    return pl.pallas_call(
        _tc_d_body,
        grid=(nb,),
        in_specs=[
            pl.BlockSpec((BN, 128), lambda i: (i, 0)),
            pl.BlockSpec((BN, 128), lambda i: (i, 0)),
            pl.BlockSpec((BN, 128), lambda i: (i, 0)),
            pl.BlockSpec((BN, 128), lambda i: (i, 0)),
            pl.BlockSpec((BN, 32), lambda i: (i, 0)),
            pl.BlockSpec((BN, 128), lambda i: (i, 0)),
            pl.BlockSpec((BN, 128), lambda i: (i, 0)),
            full((8, 256)), full((8, 128)), full((8, 256)),
            full((128, 256)), full((128, 256)), full((128, 256)),
            full((256, 256)), full((1, 256)), full((256, 128)), full((1, 128)),
            full((128, 256)), full((128, 256)), full((256, 256)), full((1, 256)),
            full((256, 256)), full((1, 256)), full((256, 128)), full((1, 128)),
            full((128, 128)), full((1, 128)), full((128, 128)), full((128, 128)),
        ],
        out_specs=[
            pl.BlockSpec((BN, 128), lambda i: (i, 0)),
            pl.BlockSpec((BN, 128), lambda i: (i, 0)),
            pl.BlockSpec((8, 128), lambda i: (0, 0)),
            pl.BlockSpec((8, 128), lambda i: (0, 0)),
        ],
        out_shape=[
            jax.ShapeDtypeStruct((N, 128), jnp.float32),
            jax.ShapeDtypeStruct((N, 128), jnp.float32),
            jax.ShapeDtypeStruct((8, 128), jnp.float32),
            jax.ShapeDtypeStruct((8, 128), jnp.float32),
        ],
    )(sege0, sege1, segzlo, segzhi, cnt_all, x1lo, x1hi, ub, sume2, u1,
      a, ba, bb, w2, b2, w3, b3,
      wg0a, wg0b, wg0c, bg0, wg1, bg1, wg2, bg2, qu, b1e, qs, qt)


def _tc_f_body(e2_ref, gsum2_ref, qe_ref, cu1_ref, out_ref):
    e3 = _relu(_mm(e2_ref[...], qe_ref[...])
               + gsum2_ref[...] + cu1_ref[0:1, :])
    m = jnp.max(e3, axis=-1, keepdims=True)
    p = jnp.exp(e3 - m)
    out_ref[...] = p / jnp.sum(p, axis=-1, keepdims=True)


def _tc_f(e2, gsum2, qe, cu1):
    nb = E // BE
    full = lambda s: pl.BlockSpec(s, lambda i: (0, 0))
    return pl.pallas_call(
        _tc_f_body,
        grid=(nb,),
        in_specs=[
            pl.BlockSpec((BE, 128), lambda i: (i, 0)),
            pl.BlockSpec((BE, 128), lambda i: (i, 0)),
            full((128, 128)), full((8, 128)),
        ],
        out_specs=[pl.BlockSpec((BE, 128), lambda i: (i, 0))],
        out_shape=[jax.ShapeDtypeStruct((E, 128), jnp.float32)],
    )(e2, gsum2, qe, cu1)[0]


_CE = 40


def _sc_gather_pair(table_a, table_b, src2d, tgt2d, width):
    info = plsc.get_sparse_core_info()
    nw = info.num_cores * info.num_subcores
    ew = E // nw
    ce = _CE
    nch = ew // ce
    nvec = width // 16
    mesh = plsc.VectorSubcoreMesh(core_axis_name="c", subcore_axis_name="s")

    @functools.partial(
        pl.kernel,
        out_type=jax.ShapeDtypeStruct((E, width), jnp.float32),
        mesh=mesh,
        compiler_params=pltpu.CompilerParams(needs_layout_passes=False),
        scratch_types=[
            pltpu.VMEM((128, ce), jnp.int32),
            pltpu.VMEM((128, ce), jnp.int32),
            pltpu.VMEM((2, ce, width), jnp.float32),
            pltpu.VMEM((2, ce, width), jnp.float32),
            pltpu.SemaphoreType.DMA,
            pltpu.SemaphoreType.DMA,
        ],
    )
    def k(ta_hbm, tb_hbm, src_hbm, tgt_hbm, out_hbm, sidx, tidx, bufa, bufb,
          sem, osem):
        wid = lax.axis_index("s") * info.num_cores + lax.axis_index("c")
        base = wid * ew
        pltpu.sync_copy(src_hbm.at[wid], sidx)
        pltpu.sync_copy(tgt_hbm.at[wid], tidx)
        pltpu.async_copy(ta_hbm.at[sidx.at[0]], bufa.at[0], sem)
        pltpu.async_copy(tb_hbm.at[tidx.at[0]], bufb.at[0], sem)

        def chunk(k_, _):
            p = lax.rem(k_, 2)

            @pl.when(k_ >= 1)
            def _():
                pltpu.make_async_copy(bufa.at[p],
                                      out_hbm.at[pl.ds(0, ce)], osem).wait()

            @pl.when(k_ + 1 < nch)
            def _():
                q = lax.rem(k_ + 1, 2)
                pltpu.async_copy(ta_hbm.at[sidx.at[k_ + 1]], bufa.at[q], sem)
                pltpu.async_copy(tb_hbm.at[tidx.at[k_ + 1]], bufb.at[q], sem)

            pltpu.make_async_copy(ta_hbm.at[pl.ds(0, ce)], bufa.at[p], sem).wait()
            pltpu.make_async_copy(tb_hbm.at[pl.ds(0, ce)], bufb.at[p], sem).wait()

            def row(r_, _):
                for j in range(nvec):
                    bufa[p, r_, pl.ds(j * 16, 16)] = (
                        bufa[p, r_, pl.ds(j * 16, 16)]
                        + bufb[p, r_, pl.ds(j * 16, 16)])
                return ()

            lax.fori_loop(0, ce, row, (), unroll=False)
            pltpu.async_copy(bufa.at[p], out_hbm.at[pl.ds(base + k_ * ce, ce)],
                             osem)
            return ()

        lax.fori_loop(0, nch, chunk, (), unroll=False)
        pltpu.make_async_copy(bufa.at[0], out_hbm.at[pl.ds(0, ce)], osem).wait()

    return k(table_a, table_b, src2d, tgt2d)


NC = 10016


def _sc_gather_pair_cnt(table_a, table_b, src2d, tgt2d, width, srcc):
    info = plsc.get_sparse_core_info()
    nw = info.num_cores * info.num_subcores
    ew = E // nw
    ce = _CE
    nch = ew // ce
    nvec = width // 16
    mesh = plsc.VectorSubcoreMesh(core_axis_name="c", subcore_axis_name="s")

    @functools.partial(
        pl.kernel,
        out_type=[
            jax.ShapeDtypeStruct((E, width), jnp.float32),
            jax.ShapeDtypeStruct((nw, NC), jnp.float32),
        ],
        mesh=mesh,
        compiler_params=pltpu.CompilerParams(needs_layout_passes=False),
        scratch_types=[
            pltpu.VMEM((128, ce), jnp.int32),
            pltpu.VMEM((128, ce), jnp.int32),
            pltpu.VMEM((40, 128), jnp.int32),
            pltpu.VMEM((NC,), jnp.float32),
            pltpu.VMEM((2, ce, width), jnp.float32),
            pltpu.VMEM((2, ce, width), jnp.float32),
            pltpu.SemaphoreType.DMA,
            pltpu.SemaphoreType.DMA,
        ],
    )
    def k(ta_hbm, tb_hbm, src_hbm, tgt_hbm, srcc_hbm, out_hbm, cnt_hbm,
          sidx, tidx, cidx, cntv, bufa, bufb, sem, osem):
        wid = lax.axis_index("s") * info.num_cores + lax.axis_index("c")
        base = wid * ew
        pltpu.sync_copy(src_hbm.at[wid], sidx)
        pltpu.sync_copy(tgt_hbm.at[wid], tidx)
        pltpu.sync_copy(srcc_hbm.at[wid], cidx)
        pltpu.async_copy(ta_hbm.at[sidx.at[0]], bufa.at[0], sem)
        pltpu.async_copy(tb_hbm.at[tidx.at[0]], bufb.at[0], sem)

        zeros16 = jnp.zeros((16,), jnp.float32)

        def zz(i, _):
            cntv[pl.ds(i * 16, 16)] = zeros16
            return ()

        lax.fori_loop(0, NC // 16, zz, (), unroll=False)
        ones16 = jnp.ones((16,), jnp.float32)

        def crow(r_, _):
            for j in range(8):
                iv = cidx[r_, pl.ds(j * 16, 16)]
                plsc.addupdate_scatter(cntv, [iv], ones16)
            return ()

        lax.fori_loop(0, 40, crow, (), unroll=False)
        pltpu.sync_copy(cntv, cnt_hbm.at[wid])

        def chunk(k_, _):
            p = lax.rem(k_, 2)

            @pl.when(k_ >= 1)
            def _():
                pltpu.make_async_copy(bufa.at[p],
                                      out_hbm.at[pl.ds(0, ce)], osem).wait()

            @pl.when(k_ + 1 < nch)
            def _():
                q = lax.rem(k_ + 1, 2)
                pltpu.async_copy(ta_hbm.at[sidx.at[k_ + 1]], bufa.at[q], sem)
                pltpu.async_copy(tb_hbm.at[tidx.at[k_ + 1]], bufb.at[q], sem)

            pltpu.make_async_copy(ta_hbm.at[pl.ds(0, ce)], bufa.at[p], sem).wait()
            pltpu.make_async_copy(tb_hbm.at[pl.ds(0, ce)], bufb.at[p], sem).wait()

            def row(r_, _):
                for j in range(nvec):
                    bufa[p, r_, pl.ds(j * 16, 16)] = (
                        bufa[p, r_, pl.ds(j * 16, 16)]
                        + bufb[p, r_, pl.ds(j * 16, 16)])
                return ()

            lax.fori_loop(0, ce, row, (), unroll=False)
            pltpu.async_copy(bufa.at[p], out_hbm.at[pl.ds(base + k_ * ce, ce)],
                             osem)
            return ()

        lax.fori_loop(0, nch, chunk, (), unroll=False)
        pltpu.make_async_copy(bufa.at[0], out_hbm.at[pl.ds(0, ce)], osem).wait()

    return k(table_a, table_b, src2d, tgt2d, srcc)


NP = 10112


def _sc_segment_xc(xclo, xchi, srcst, tgtst, zx):
    info = plsc.get_sparse_core_info()
    ns = info.num_subcores
    ew = E // ns
    ce = 80
    stripe = NP // ns
    mesh = plsc.VectorSubcoreMesh(core_axis_name="c", subcore_axis_name="s")

    nch = ew // ce

    @functools.partial(
        pl.kernel,
        out_type=[
            jax.ShapeDtypeStruct((NP, 128), jnp.float32),
            jax.ShapeDtypeStruct((NP, 128), jnp.float32),
        ],
        mesh=mesh,
        compiler_params=pltpu.CompilerParams(needs_layout_passes=False),
        scratch_types=[
            pltpu.VMEM_SHARED((NP, 128), jnp.float32),
            pltpu.VMEM((2, ce), jnp.int32),
            pltpu.VMEM((2, ce), jnp.int32),
            pltpu.VMEM((2, ce, 128), jnp.float32),
            pltpu.SemaphoreType.DMA,
            pltpu.SemaphoreType.DMA,
        ],
    )
    def k(xclo_hbm, xchi_hbm, srcst_hbm, tgtst_hbm, zx_hbm,
          seglo_hbm, seghi_hbm, accz, sidxb, tidxb, bufg, isem, gsem):
        cid = lax.axis_index("c")
        sid = lax.axis_index("s")
        row0 = sid * stripe
        pltpu.sync_copy(zx_hbm.at[pl.ds(row0, stripe)],
                        accz.at[pl.ds(row0, stripe)])
        plsc.subcore_barrier()

        def body(xc_hbm):
            pltpu.sync_copy(srcst_hbm.at[sid, 0], sidxb.at[0])
            pltpu.sync_copy(tgtst_hbm.at[sid, 0], tidxb.at[0])
            pltpu.async_copy(xc_hbm.at[tidxb.at[0]], bufg.at[0], gsem)
            pltpu.async_copy(srcst_hbm.at[sid, 1], sidxb.at[1], isem)
            pltpu.async_copy(tgtst_hbm.at[sid, 1], tidxb.at[1], isem)

            def chunk(k_, _):
                p = lax.rem(k_, 2)
                q = lax.rem(k_ + 1, 2)

                @pl.when(k_ + 1 < nch)
                def _():
                    pltpu.make_async_copy(srcst_hbm.at[sid, 0], sidxb.at[q],
                                          isem).wait()
                    pltpu.make_async_copy(tgtst_hbm.at[sid, 0], tidxb.at[q],
                                          isem).wait()
                    pltpu.async_copy(xc_hbm.at[tidxb.at[q]], bufg.at[q], gsem)

                pltpu.make_async_copy(xc_hbm.at[pl.ds(0, ce)], bufg.at[p],
                                      gsem).wait()
                pltpu.sync_copy(bufg.at[p], accz.at[sidxb.at[p]], add=True)

                @pl.when(k_ + 2 < nch)
                def _():
                    pltpu.async_copy(srcst_hbm.at[sid, k_ + 2], sidxb.at[p],
                                     isem)
                    pltpu.async_copy(tgtst_hbm.at[sid, k_ + 2], tidxb.at[p],
                                     isem)

                return ()

            lax.fori_loop(0, nch, chunk, (), unroll=False)

        @pl.when(cid == 0)
        def _():
            body(xclo_hbm)

        @pl.when(cid == 1)
        def _():
            body(xchi_hbm)

        plsc.subcore_barrier()

        @pl.when(cid == 0)
        def _():
            pltpu.sync_copy(accz.at[pl.ds(row0, stripe)],
                            seglo_hbm.at[pl.ds(row0, stripe)])

        @pl.when(cid == 1)
        def _():
            pltpu.sync_copy(accz.at[pl.ds(row0, stripe)],
                            seghi_hbm.at[pl.ds(row0, stripe)])

    return k(xclo, xchi, srcst, tgtst, zx)


def _sc_segment_e(e2, src, zx):
    info = plsc.get_sparse_core_info()
    nc, ns = info.num_cores, info.num_subcores
    ew = E // (nc * ns)
    ce = _CE
    stripe = NP // ns
    mesh = plsc.VectorSubcoreMesh(core_axis_name="c", subcore_axis_name="s")

    nch = ew // ce

    @functools.partial(
        pl.kernel,
        out_type=[
            jax.ShapeDtypeStruct((NP, 128), jnp.float32),
            jax.ShapeDtypeStruct((NP, 128), jnp.float32),
        ],
        mesh=mesh,
        compiler_params=pltpu.CompilerParams(needs_layout_passes=False),
        scratch_types=[
            pltpu.VMEM_SHARED((NP, 128), jnp.float32),
            pltpu.VMEM((128, ce), jnp.int32),
            pltpu.VMEM((2, ce, 128), jnp.float32),
            pltpu.SemaphoreType.DMA,
        ],
    )
    def k(e2_hbm, src_hbm, zx_hbm, sege0_hbm, sege1_hbm,
          acce, sidx, bufe, sem):
        cid = lax.axis_index("c")
        sid = lax.axis_index("s")
        row0 = sid * stripe
        pltpu.sync_copy(zx_hbm.at[pl.ds(row0, stripe)],
                        acce.at[pl.ds(row0, stripe)])
        wid = cid * ns + sid
        pltpu.sync_copy(src_hbm.at[wid], sidx)
        plsc.subcore_barrier()

        base = wid * ew
        pltpu.async_copy(e2_hbm.at[pl.ds(base, ce)], bufe.at[0], sem)

        def chunk(k_, _):
            p = lax.rem(k_, 2)
            pltpu.make_async_copy(e2_hbm.at[pl.ds(0, ce)], bufe.at[p],
                                  sem).wait()

            @pl.when(k_ + 1 < nch)
            def _():
                q = lax.rem(k_ + 1, 2)
                pltpu.async_copy(e2_hbm.at[pl.ds(base + (k_ + 1) * ce, ce)],
                                 bufe.at[q], sem)

            pltpu.sync_copy(bufe.at[p], acce.at[sidx.at[k_]], add=True)
            return ()

        lax.fori_loop(0, nch, chunk, (), unroll=False)
        plsc.subcore_barrier()

        @pl.when(cid == 0)
        def _():
            pltpu.sync_copy(acce.at[pl.ds(row0, stripe)],
                            sege0_hbm.at[pl.ds(row0, stripe)])

        @pl.when(cid == 1)
        def _():
            pltpu.sync_copy(acce.at[pl.ds(row0, stripe)],
                            sege1_hbm.at[pl.ds(row0, stripe)])

    return k(e2, src, zx)


def kernel(x, edge_index, e, params):
    f32 = jnp.float32
    src = edge_index[:, 0]
    tgt = edge_index[:, 1]
    r = lambda b: b.reshape(1, -1)

    wn, bn = params["emb_node"]
    we_, be_ = params["emb_edge"]
    wg, bg = params["emb_glob"]
    w0e, b0e = params["gnn0_edge"]
    w0n, b0n = params["gnn0_node"]
    w0g, b0g = params["gnn0_glob"]
    w1e, b1e = params["gnn1_edge"]

    W1 = w0e[0]
    We, Ws, Wt, Wu = W1[0:256], W1[256:512], W1[512:768], W1[768:1024]
    Wn1 = w0n[0]
    A, B, C, U = Wn1[0:128], Wn1[128:384], Wn1[384:640], Wn1[640:896]
    Q = w1e[0]
    Qe, Qs, Qt, Qu = Q[0:128], Q[128:256], Q[256:384], Q[384:512]
    Wg0 = w0g[0]

    def idx3(a, nw, ce, rows):
        a3 = a.reshape(nw, -1, ce)
        return jnp.pad(a3, ((0, 0), (0, rows - a3.shape[1]), (0, 0)))

    src40 = idx3(src, 32, _CE, 128)
    tgt40 = idx3(tgt, 32, _CE, 128)
    srcst = src.reshape(16, -1, 80)
    tgtst = tgt.reshape(16, -1, 80)
    srcp = jnp.pad(src.reshape(32, 5000), ((0, 0), (0, 120)),
                   constant_values=NC - 8).reshape(32, 40, 128)

    x1lo, x1hi, xs, xt, xclo, xchi, u1, cu0, ub = _tc_a(
        x, wn[0], r(bn[0]), wn[1], r(bn[1]), Ws, Wt, C,
        r(bg[0]), wg[1], r(bg[1]), Wu, r(b0e[0]), U, r(b0n[0]))

    gsum, cnt_pad = _sc_gather_pair_cnt(xs, xt, src40, tgt40, 256, srcp)
    cnt_all = cnt_pad[:, :N]

    zx = jnp.zeros((NP, 128), f32)
    segzlo, segzhi = _sc_segment_xc(xclo, xchi, srcst, tgtst, zx)

    e2, sume2 = _tc_b(
        e, gsum, we_[0], r(be_[0]), we_[1], r(be_[1]), We, cu0,
        w0e[1], r(b0e[1]), w0e[2], r(b0e[2]))

    sege0, sege1 = _sc_segment_e(e2, src40, zx)

    xs2, xt2, cu1, _sumx2 = _tc_d(
        sege0[:N], sege1[:N], segzlo[:N], segzhi[:N], cnt_all.T,
        x1lo, x1hi, ub, sume2, u1,
        A, B[:128], B[128:],
        w0n[1], r(b0n[1]), w0n[2], r(b0n[2]),
        Wg0[0:128], Wg0[128:256], Wg0[256:512], r(b0g[0]),
        w0g[1], r(b0g[1]), w0g[2], r(b0g[2]), Qu, r(b1e[0]), Qs, Qt)

    gsum2 = _sc_gather_pair(xs2, xt2, src40, tgt40, 128)

    return _tc_f(e2, gsum2, Qe, cu1)
